# Initial kernel scaffold; baseline (speedup 1.0000x reference)
import jax
import jax.numpy as jnp
from jax.experimental import pallas as pl


def kernel(x, edge_index, batch, W0, a_src0, a_dst0, b0, W1, a_src1, a_dst1, b1, W2, a_src2, a_dst2, b2, W3, a_src3, a_dst3, b3, Wf, bf):
    raise NotImplementedError("write your pallas kernel here")


# speedup vs baseline: 76.7574x; 76.7574x over previous
import functools

import jax
import jax.numpy as jnp
from jax import lax
from jax.experimental import pallas as pl
from jax.experimental.pallas import tpu as pltpu
from jax.experimental.pallas import tpu_sc as plsc

N = 10000
E = 320000
D_IN = 128
HID = 64
H = 4
C = 16
OUT = 5
G = 128
ALW = 16

BLK = 256
N_PAD = 10240
NGRID = N_PAD // BLK

NC, NS = 2, 16
NW = NC * NS
ROWS_PER_TILE = N_PAD // NS
NBLK = 81
EPW = NBLK * 128
E_PAD = EPW * NW


def _sc_edge_body(h_hbm, al_hbm, src_hbm, dst_hbm, acc_out, s_out,
                  sidx_v, didx_v, sidx2_v, didx2_v, hsrc_v, w_v, asrc_v,
                  adst_v, acc_sh, s_sh, gsem):
    cid = lax.axis_index("c")
    sid = lax.axis_index("s")
    wid = sid * NC + cid

    pltpu.sync_copy(src_hbm.at[pl.ds(wid * EPW, EPW)], sidx_v)
    pltpu.sync_copy(dst_hbm.at[pl.ds(wid * EPW, EPW)], didx_v)

    zero16 = jnp.zeros((16,), jnp.float32)

    @pl.loop(0, 128)
    def _zero_bufs(e):
        for h in range(H):
            hsrc_v[e, pl.ds(h * 16, 16)] = zero16
        w_v[e, pl.ds(0, 16)] = zero16

    for r in range(ROWS_PER_TILE // 128):
        base = sid * ROWS_PER_TILE + r * 128
        pltpu.sync_copy(hsrc_v, acc_sh.at[pl.ds(base, 128)])
        pltpu.sync_copy(w_v, s_sh.at[pl.ds(base, 128)])

    plsc.subcore_barrier()

    iota16 = lax.iota(jnp.int32, 16)

    @pl.loop(0, NBLK)
    def _edge_block(j):
        for g in range(8):
            sidx2_v[0, pl.ds(g * 16, 16)] = sidx_v[pl.ds(j * 128 + g * 16, 16)]
            didx2_v[0, pl.ds(g * 16, 16)] = didx_v[pl.ds(j * 128 + g * 16, 16)]

        sidx_row = sidx2_v.at[0]
        didx_row = didx2_v.at[0]
        d1 = pltpu.async_copy(h_hbm.at[sidx_row], hsrc_v, gsem)
        d2 = pltpu.async_copy(al_hbm.at[sidx_row], asrc_v, gsem)
        d3 = pltpu.async_copy(al_hbm.at[didx_row], adst_v, gsem)
        d1.wait()
        d2.wait()
        d3.wait()

        for g in range(8):
            rows = iota16 + g * 16
            for h in range(H):
                colh = jnp.full((16,), h, jnp.int32)
                a_s = plsc.load_gather(asrc_v, [rows, colh])
                a_d = plsc.load_gather(adst_v, [rows, colh + H])
                e = a_s + a_d
                e = jnp.where(e >= 0.0, e, 0.2 * e)
                plsc.store_scatter(w_v, [rows, colh], jnp.exp(e))

        @pl.loop(0, 128)
        def _scale(e):
            wrow = w_v[e, pl.ds(0, 16)]
            for h in range(H):
                hsrc_v[e, pl.ds(h * 16, 16)] = (
                    hsrc_v[e, pl.ds(h * 16, 16)] * wrow[h])

        pltpu.sync_copy(w_v, s_sh.at[didx_row], add=True)
        pltpu.sync_copy(hsrc_v, acc_sh.at[didx_row], add=True)

    plsc.subcore_barrier()

    for r in range(ROWS_PER_TILE // 128):
        base = sid * ROWS_PER_TILE + r * 128
        pltpu.sync_copy(acc_sh.at[pl.ds(base, 128)], hsrc_v)
        pltpu.sync_copy(hsrc_v, acc_out.at[cid, pl.ds(base, 128)])
        pltpu.sync_copy(s_sh.at[pl.ds(base, 128)], w_v)
        pltpu.sync_copy(w_v, s_out.at[cid, pl.ds(base, 128)])


def _sc_edge(h, al, src_r, dst_r):
    mesh = plsc.VectorSubcoreMesh(core_axis_name="c", subcore_axis_name="s")
    return pl.kernel(
        _sc_edge_body,
        out_type=(
            jax.ShapeDtypeStruct((NC, N_PAD, HID), jnp.float32),
            jax.ShapeDtypeStruct((NC, N_PAD, 16), jnp.float32),
        ),
        mesh=mesh,
        scratch_types=[
            pltpu.VMEM((EPW,), jnp.int32),
            pltpu.VMEM((EPW,), jnp.int32),
            pltpu.VMEM((1, 128), jnp.int32),
            pltpu.VMEM((1, 128), jnp.int32),
            pltpu.VMEM((128, HID), jnp.float32),
            pltpu.VMEM((128, 16), jnp.float32),
            pltpu.VMEM((128, ALW), jnp.float32),
            pltpu.VMEM((128, ALW), jnp.float32),
            pltpu.VMEM_SHARED((N_PAD, HID), jnp.float32),
            pltpu.VMEM_SHARED((N_PAD, 16), jnp.float32),
            pltpu.SemaphoreType.DMA,
        ],
        compiler_params=pltpu.CompilerParams(
            needs_layout_passes=False,
            use_tc_tiling_on_sc=False,
        ),
    )(h, al, src_r, dst_r)


def _tc0_body(x_ref, w_ref, a_ref, h_ref, al_ref):
    h = jnp.dot(x_ref[...], w_ref[...], preferred_element_type=jnp.float32)
    h_ref[...] = h
    al_ref[...] = jnp.dot(h, a_ref[...], preferred_element_type=jnp.float32)


def _tc0(x, W, A):
    return pl.pallas_call(
        _tc0_body,
        grid=(NGRID,),
        in_specs=[
            pl.BlockSpec((BLK, D_IN), lambda i: (i, 0)),
            pl.BlockSpec((D_IN, HID), lambda i: (0, 0)),
            pl.BlockSpec((HID, ALW), lambda i: (0, 0)),
        ],
        out_specs=[
            pl.BlockSpec((BLK, HID), lambda i: (i, 0)),
            pl.BlockSpec((BLK, ALW), lambda i: (i, 0)),
        ],
        out_shape=[
            jax.ShapeDtypeStruct((N_PAD, HID), jnp.float32),
            jax.ShapeDtypeStruct((N_PAD, ALW), jnp.float32),
        ],
    )(x, W, A)


def _tcmid_body(acc_ref, s_ref, b_ref, k_ref, w_ref, a_ref, h_ref, al_ref):
    accs = acc_ref[0] + acc_ref[1]
    ss = s_ref[0] + s_ref[1]
    recip = 1.0 / (ss + 1e-16)
    x = accs * jnp.dot(recip, k_ref[...], preferred_element_type=jnp.float32)
    x = x + b_ref[...]
    h = jnp.dot(x, w_ref[...], preferred_element_type=jnp.float32)
    h_ref[...] = h
    al_ref[...] = jnp.dot(h, a_ref[...], preferred_element_type=jnp.float32)


def _tcmid(acc, s, b2d, Kpad, W, A):
    return pl.pallas_call(
        _tcmid_body,
        grid=(NGRID,),
        in_specs=[
            pl.BlockSpec((NC, BLK, HID), lambda i: (0, i, 0)),
            pl.BlockSpec((NC, BLK, 16), lambda i: (0, i, 0)),
            pl.BlockSpec((1, HID), lambda i: (0, 0)),
            pl.BlockSpec((16, HID), lambda i: (0, 0)),
            pl.BlockSpec((HID, HID), lambda i: (0, 0)),
            pl.BlockSpec((HID, ALW), lambda i: (0, 0)),
        ],
        out_specs=[
            pl.BlockSpec((BLK, HID), lambda i: (i, 0)),
            pl.BlockSpec((BLK, ALW), lambda i: (i, 0)),
        ],
        out_shape=[
            jax.ShapeDtypeStruct((N_PAD, HID), jnp.float32),
            jax.ShapeDtypeStruct((N_PAD, ALW), jnp.float32),
        ],
    )(acc, s, b2d, Kpad, W, A)


def _tcfinal_body(acc_ref, s_ref, b_ref, k_ref, batch_ref, wf_ref, bf_ref,
                  out_ref, accum):
    i = pl.program_id(0)

    @pl.when(i == 0)
    def _():
        accum[...] = jnp.zeros_like(accum)

    accs = acc_ref[0] + acc_ref[1]
    ss = s_ref[0] + s_ref[1]
    recip = 1.0 / (ss + 1e-16)
    x4 = accs * jnp.dot(recip, k_ref[...], preferred_element_type=jnp.float32)
    x4 = x4 + b_ref[...]
    bb = batch_ref[0, 0, :]
    onehot = (lax.broadcasted_iota(jnp.int32, (G, BLK), 0)
              == bb[None, :]).astype(jnp.float32)
    haug = jnp.concatenate(
        [x4, jnp.ones((BLK, 1), jnp.float32), jnp.zeros((BLK, 63), jnp.float32)],
        axis=1)
    accum[...] += jnp.dot(onehot, haug, preferred_element_type=jnp.float32)

    @pl.when(i == pl.num_programs(0) - 1)
    def _():
        sums = accum[:, :HID]
        cnt = accum[:, HID:HID + 1]
        pooled = sums / jnp.maximum(cnt, 1.0)
        out_ref[...] = (jnp.dot(pooled, wf_ref[...],
                                preferred_element_type=jnp.float32)
                        + bf_ref[...])


def _tcfinal(acc, s, b2d, Kpad, batch_r, Wfp, bfp):
    return pl.pallas_call(
        _tcfinal_body,
        grid=(NGRID,),
        in_specs=[
            pl.BlockSpec((NC, BLK, HID), lambda i: (0, i, 0)),
            pl.BlockSpec((NC, BLK, 16), lambda i: (0, i, 0)),
            pl.BlockSpec((1, HID), lambda i: (0, 0)),
            pl.BlockSpec((16, HID), lambda i: (0, 0)),
            pl.BlockSpec((1, 1, BLK), lambda i: (i, 0, 0)),
            pl.BlockSpec((HID, 128), lambda i: (0, 0)),
            pl.BlockSpec((1, 128), lambda i: (0, 0)),
        ],
        out_specs=pl.BlockSpec((G, 128), lambda i: (0, 0)),
        out_shape=jax.ShapeDtypeStruct((G, 128), jnp.float32),
        scratch_shapes=[pltpu.VMEM((G, 128), jnp.float32)],
    )(acc, s, b2d, Kpad, batch_r, Wfp, bfp)


def _attn_mat(a_src, a_dst):
    eye = jnp.eye(H, dtype=jnp.float32)
    As = (eye[:, None, :] * a_src[:, :, None]).reshape(HID, H)
    Ad = (eye[:, None, :] * a_dst[:, :, None]).reshape(HID, H)
    return jnp.concatenate(
        [As, Ad, jnp.zeros((HID, ALW - 2 * H), jnp.float32)], axis=1)


def kernel(x, edge_index, batch, W0, a_src0, a_dst0, b0, W1, a_src1, a_dst1,
           b1, W2, a_src2, a_dst2, b2, W3, a_src3, a_dst3, b3, Wf, bf):
    f32 = jnp.float32
    x_pad = jnp.zeros((N_PAD, D_IN), f32).at[:N].set(x.astype(f32))

    loops = jnp.arange(N, dtype=jnp.int32)
    pad_e = E_PAD - (E + N)
    src_r = jnp.concatenate([edge_index[0].astype(jnp.int32), loops,
                             jnp.full((pad_e,), N, jnp.int32)])
    dst_r = jnp.concatenate([edge_index[1].astype(jnp.int32), loops,
                             jnp.full((pad_e,), N, jnp.int32)])

    batch_r = (jnp.full((N_PAD,), G + 7, jnp.int32)
               .at[:N].set(batch.astype(jnp.int32))
               .reshape(NGRID, 1, BLK))

    Kpad = jnp.zeros((16, HID), f32).at[:H].set(
        jnp.repeat(jnp.eye(H, dtype=f32), C, axis=1))
    Wfp = jnp.zeros((HID, 128), f32).at[:, :OUT].set(Wf)
    bfp = jnp.zeros((1, 128), f32).at[0, :OUT].set(bf)

    A = [_attn_mat(a_src0, a_dst0), _attn_mat(a_src1, a_dst1),
         _attn_mat(a_src2, a_dst2), _attn_mat(a_src3, a_dst3)]
    Ws = [W1, W2, W3]
    bs = [b0.reshape(1, HID), b1.reshape(1, HID), b2.reshape(1, HID),
          b3.reshape(1, HID)]

    h, al = _tc0(x_pad, W0, A[0])
    for i in range(3):
        acc, s = _sc_edge(h, al, src_r, dst_r)
        h, al = _tcmid(acc, s, bs[i], Kpad, Ws[i], A[i + 1])
    acc, s = _sc_edge(h, al, src_r, dst_r)
    out = _tcfinal(acc, s, bs[3], Kpad, batch_r, Wfp, bfp)
    return out[:, :OUT]
